# Initial kernel scaffold; baseline (speedup 1.0000x reference)
import jax
import jax.numpy as jnp
from jax.experimental import pallas as pl


def kernel(X_n, snorm, Eemb0, Eemb1, Eemb2, W0, b0, W1, b1, W2, b2, Wm1, bm1, Wm2, bm2, edge_index, edge_attr, batch):
    raise NotImplementedError("write your pallas kernel here")


# speedup vs baseline: 2.3481x; 2.3481x over previous
import math

import jax
import jax.numpy as jnp
from jax import lax
from jax.experimental import pallas as pl
from jax.experimental.pallas import tpu as pltpu
from jax.experimental.pallas import tpu_sc as plsc

_DEG_HIST = (0.0, 500.0, 1500.0, 2500.0, 2500.0, 1500.0, 1000.0, 500.0)
_DELTA = sum(h * math.log(i + 1.0) for i, h in enumerate(_DEG_HIST)) / sum(_DEG_HIST)

_B = 64
_PREC = lax.Precision.HIGHEST
_G = 128
_CE = 128
_L = 16


def _sc_edge_stage(h, eemb, perm_p, sdst_p, src_u, attr_u, bounds, n_chunks, n_pad):
    D = h.shape[1]
    nsub = D // _L
    info = plsc.get_sparse_core_info()
    NC, NS = info.num_cores, info.num_subcores
    NW = NC * NS
    kmax = -(-n_chunks // NW)
    nb_len = bounds.shape[0]
    mesh = plsc.VectorSubcoreMesh(core_axis_name="c", subcore_axis_name="s")

    def body(h_hbm, eemb_hbm, perm_hbm, sdst_hbm, src_hbm, attr_hbm, bnd_hbm,
             s_hbm, q_hbm, mx_hbm, mn_hbm,
             eemb_v, bnd_v, permc, srcc, attrc, dstc, rows,
             acc_s, acc_q, acc_mx, acc_mn, sem):
        wid = lax.axis_index("s") * NC + lax.axis_index("c")
        pltpu.sync_copy(eemb_hbm, eemb_v)
        pltpu.sync_copy(bnd_hbm, bnd_v)
        zero = jnp.zeros((_L,), jnp.float32)
        neg = jnp.full((_L,), -jnp.inf, jnp.float32)
        pos = jnp.full((_L,), jnp.inf, jnp.float32)

        for k in range(kmax):
            c = wid + NW * k

            @pl.when(c < n_chunks)
            def _():
                nb = c * _G
                bv = bnd_v[pl.ds(c, _L)]
                e0 = bv[0]
                e1 = bv[1]

                def init_row(r, carry):
                    for j in range(nsub):
                        sl = pl.ds(j * _L, _L)
                        acc_s[r, sl] = zero
                        acc_q[r, sl] = zero
                        acc_mx[r, sl] = neg
                        acc_mn[r, sl] = pos
                    return carry

                lax.fori_loop(0, _G, init_row, 0)

                eb0 = (e0 // 8) * 8
                nrounds = (e1 - eb0 + _CE - 1) // _CE

                def round_body(i, carry):
                    eb = eb0 + i * _CE
                    pltpu.sync_copy(perm_hbm.at[pl.ds(eb, _CE)], permc)
                    pltpu.sync_copy(sdst_hbm.at[pl.ds(eb, _CE)],
                                    dstc.at[pl.ds(0, _CE)])
                    pltpu.async_copy(src_hbm.at[permc], srcc, sem).wait()
                    pltpu.async_copy(attr_hbm.at[permc],
                                     attrc.at[pl.ds(0, _CE)], sem).wait()
                    pltpu.async_copy(h_hbm.at[srcc], rows, sem).wait()
                    lo = jnp.maximum(e0 - eb, 0)
                    hi = jnp.minimum(e1 - eb, _CE)

                    def edge_body(e, ecarry):
                        ldst = dstc[pl.ds(e, _L)][0] - nb
                        at = attrc[pl.ds(e, _L)][0]
                        for j in range(nsub):
                            sl = pl.ds(j * _L, _L)
                            m = rows[e, sl] + eemb_v[at, sl]
                            acc_s[ldst, sl] = acc_s[ldst, sl] + m
                            acc_q[ldst, sl] = acc_q[ldst, sl] + m * m
                            acc_mx[ldst, sl] = jnp.maximum(acc_mx[ldst, sl], m)
                            acc_mn[ldst, sl] = jnp.minimum(acc_mn[ldst, sl], m)
                        return ecarry

                    lax.fori_loop(lo, hi, edge_body, 0)
                    return carry

                lax.fori_loop(0, nrounds, round_body, 0)
                pltpu.sync_copy(acc_s, s_hbm.at[pl.ds(nb, _G)])
                pltpu.sync_copy(acc_q, q_hbm.at[pl.ds(nb, _G)])
                pltpu.sync_copy(acc_mx, mx_hbm.at[pl.ds(nb, _G)])
                pltpu.sync_copy(acc_mn, mn_hbm.at[pl.ds(nb, _G)])

    out = jax.ShapeDtypeStruct((n_pad, D), jnp.float32)
    fn = pl.kernel(
        body,
        out_type=(out, out, out, out),
        mesh=mesh,
        scratch_types=[
            pltpu.VMEM((eemb.shape[0], D), jnp.float32),
            pltpu.VMEM((nb_len,), jnp.int32),
            pltpu.VMEM((_CE,), jnp.int32),
            pltpu.VMEM((_CE,), jnp.int32),
            pltpu.VMEM((_CE + _L,), jnp.int32),
            pltpu.VMEM((_CE + _L,), jnp.int32),
            pltpu.VMEM((_CE, D), jnp.float32),
            pltpu.VMEM((_G, D), jnp.float32),
            pltpu.VMEM((_G, D), jnp.float32),
            pltpu.VMEM((_G, D), jnp.float32),
            pltpu.VMEM((_G, D), jnp.float32),
            pltpu.SemaphoreType.DMA,
        ],
    )
    return fn(h, eemb, perm_p, sdst_p, src_u, attr_u, bounds)


def _tc_layer(h, s, q, mx, mn, rs0, rs1, sn2, W, b2):
    N, D = h.shape
    BN = 400
    grid = (N // BN,)

    def body(h_r, s_r, q_r, mx_r, mn_r, r0_r, r1_r, sn_r, w_r, b_r, o_r):
        deg = (r1_r[...] - r0_r[...]).astype(jnp.float32)
        degc = jnp.maximum(deg, 1.0)
        logd = jnp.log(deg + 1.0)
        amp = logd / _DELTA
        att = _DELTA / jnp.maximum(logd, 1e-6)
        hv = h_r[...]
        mean = s_r[...] / degc
        std = jnp.sqrt(jnp.maximum(q_r[...] / degc - mean * mean, 0.0) + 1e-5)
        nonempty = deg > 0.0
        mxv = jnp.where(nonempty, mx_r[...], 0.0)
        mnv = jnp.where(nonempty, mn_r[...], 0.0)
        agg = jnp.concatenate([mean, mnv, mxv, std], axis=1)
        z = jnp.concatenate([hv, agg, agg * amp, agg * att], axis=1)
        o = jnp.dot(z, w_r[...], precision=_PREC,
                    preferred_element_type=jnp.float32) + b_r[...]
        o = jnp.maximum(o, 0.0) + hv
        o_r[...] = o * sn_r[...]

    row_spec = pl.BlockSpec((BN, D), lambda i: (i, 0))
    col_spec = pl.BlockSpec((BN, 1), lambda i: (i, 0))
    return pl.pallas_call(
        body,
        grid=grid,
        in_specs=[
            row_spec, row_spec, row_spec, row_spec, row_spec,
            col_spec, col_spec, col_spec,
            pl.BlockSpec(W.shape, lambda i: (0, 0)),
            pl.BlockSpec((1, D), lambda i: (0, 0)),
        ],
        out_specs=row_spec,
        out_shape=jax.ShapeDtypeStruct((N, D), jnp.float32),
    )(h, s, q, mx, mn, rs0, rs1, sn2, W, b2)


def _tc_pool(h, batch2, Wm1, bm1_2, Wm2, bm2_2):
    N, D = h.shape
    BN = 400
    g = N // BN

    def body(h_r, b_r, w1_r, b1_r, w2_r, b2_r, o_r, acc):
        i = pl.program_id(0)

        @pl.when(i == 0)
        def _():
            acc[...] = jnp.zeros_like(acc)

        oh = (b_r[...] == lax.broadcasted_iota(jnp.int32, (BN, _B), 1))
        acc[...] += lax.dot_general(
            oh.astype(jnp.float32), h_r[...],
            (((0,), (0,)), ((), ())), precision=_PREC,
            preferred_element_type=jnp.float32)

        @pl.when(i == g - 1)
        def _():
            p = acc[...]
            hid = jnp.maximum(
                jnp.dot(p, w1_r[...], precision=_PREC,
                        preferred_element_type=jnp.float32)
                + b1_r[...], 0.0)
            o_r[...] = (jnp.dot(hid, w2_r[...], precision=_PREC,
                                preferred_element_type=jnp.float32) + b2_r[...])

    return pl.pallas_call(
        body,
        grid=(g,),
        in_specs=[
            pl.BlockSpec((BN, D), lambda i: (i, 0)),
            pl.BlockSpec((BN, 1), lambda i: (i, 0)),
            pl.BlockSpec(Wm1.shape, lambda i: (0, 0)),
            pl.BlockSpec((1, D), lambda i: (0, 0)),
            pl.BlockSpec(Wm2.shape, lambda i: (0, 0)),
            pl.BlockSpec((1, 1), lambda i: (0, 0)),
        ],
        out_specs=pl.BlockSpec((_B, 1), lambda i: (0, 0)),
        out_shape=jax.ShapeDtypeStruct((_B, 1), jnp.float32),
        scratch_shapes=[pltpu.VMEM((_B, D), jnp.float32)],
    )(h, batch2, Wm1, bm1_2, Wm2, bm2_2)


def kernel(X_n, snorm, Eemb0, Eemb1, Eemb2, W0, b0, W1, b1, W2, b2,
           Wm1, bm1, Wm2, bm2, edge_index, edge_attr, batch):
    N, D = X_n.shape
    E = edge_index.shape[1]
    src = edge_index[0]
    dst = edge_index[1]

    sdst, perm = lax.sort_key_val(dst, jnp.arange(E, dtype=jnp.int32))
    row_start = jnp.searchsorted(sdst, jnp.arange(N + 1)).astype(jnp.int32)

    n_chunks = -(-N // _G)
    n_pad = n_chunks * _G
    nb_len = -(-(n_chunks + _L) // _L) * _L
    cidx = jnp.minimum(jnp.arange(n_chunks + 1) * _G, N)
    bounds = jnp.concatenate(
        [row_start[cidx],
         jnp.full((nb_len - (n_chunks + 1),), E, jnp.int32)])
    pad_i = jnp.zeros((_CE + 8,), jnp.int32)
    perm_p = jnp.concatenate([perm, pad_i])
    sdst_p = jnp.concatenate([sdst, pad_i])

    rs0 = row_start[:N].reshape(N, 1)
    rs1 = row_start[1:N + 1].reshape(N, 1)
    sn2 = snorm.reshape(N, 1)

    h = X_n
    for Eemb, W, b in ((Eemb0, W0, b0), (Eemb1, W1, b1), (Eemb2, W2, b2)):
        s, q, mx, mn = _sc_edge_stage(h, Eemb, perm_p, sdst_p, src,
                                      edge_attr, bounds, n_chunks, n_pad)
        h = _tc_layer(h, s[:N], q[:N], mx[:N], mn[:N], rs0, rs1, sn2,
                      W, b.reshape(1, D))

    y = _tc_pool(h, batch.reshape(N, 1), Wm1, bm1.reshape(1, D),
                 Wm2, bm2.reshape(1, 1))
    return y.reshape(_B)


# speedup vs baseline: 2.4626x; 1.0488x over previous
import math

import jax
import jax.numpy as jnp
from jax import lax
from jax.experimental import pallas as pl
from jax.experimental.pallas import tpu as pltpu
from jax.experimental.pallas import tpu_sc as plsc

_DEG_HIST = (0.0, 500.0, 1500.0, 2500.0, 2500.0, 1500.0, 1000.0, 500.0)
_DELTA = sum(h * math.log(i + 1.0) for i, h in enumerate(_DEG_HIST)) / sum(_DEG_HIST)

_B = 64
_PREC = lax.Precision.HIGHEST
_G = 128
_CE = 128
_L = 16


def _sc_edge_stage(h, eemb, ssrc_p, sattr_p, sdst_p, bounds, n_chunks, n_pad):
    D = h.shape[1]
    nsub = D // _L
    info = plsc.get_sparse_core_info()
    NC, NS = info.num_cores, info.num_subcores
    NW = NC * NS
    kmax = -(-n_chunks // NW)
    nb_len = bounds.shape[0]
    mesh = plsc.VectorSubcoreMesh(core_axis_name="c", subcore_axis_name="s")

    def body(h_hbm, eemb_hbm, ssrc_hbm, sattr_hbm, sdst_hbm, bnd_hbm,
             s_hbm, q_hbm, mx_hbm, mn_hbm,
             eemb_v, bnd_v, srcc, attrc, dstc, rows,
             acc_s, acc_q, acc_mx, acc_mn, sem):
        wid = lax.axis_index("s") * NC + lax.axis_index("c")
        pltpu.sync_copy(eemb_hbm, eemb_v)
        pltpu.sync_copy(bnd_hbm, bnd_v)
        zero = jnp.zeros((_L,), jnp.float32)
        neg = jnp.full((_L,), -jnp.inf, jnp.float32)
        pos = jnp.full((_L,), jnp.inf, jnp.float32)

        for k in range(kmax):
            c = wid + NW * k

            @pl.when(c < n_chunks)
            def _():
                nb = c * _G
                bv = bnd_v[pl.ds(c, _L)]
                e0 = bv[0]
                e1 = bv[1]

                def init_row(r, carry):
                    for j in range(nsub):
                        sl = pl.ds(j * _L, _L)
                        acc_s[r, sl] = zero
                        acc_q[r, sl] = zero
                        acc_mx[r, sl] = neg
                        acc_mn[r, sl] = pos
                    return carry

                lax.fori_loop(0, _G, init_row, 0)

                eb0 = (e0 // 8) * 8
                nrounds = (e1 - eb0 + _CE - 1) // _CE

                def round_body(i, carry):
                    eb = eb0 + i * _CE
                    pltpu.sync_copy(ssrc_hbm.at[pl.ds(eb, _CE)],
                                    srcc.at[pl.ds(0, _CE)])
                    pltpu.sync_copy(sattr_hbm.at[pl.ds(eb, _CE)],
                                    attrc.at[pl.ds(0, _CE)])
                    pltpu.sync_copy(sdst_hbm.at[pl.ds(eb, _CE)],
                                    dstc.at[pl.ds(0, _CE)])
                    pltpu.async_copy(h_hbm.at[srcc], rows, sem).wait()
                    lo = jnp.maximum(e0 - eb, 0)
                    hi = jnp.minimum(e1 - eb, _CE)

                    def edge_body(e, ecarry):
                        ldst = dstc[pl.ds(e, _L)][0] - nb
                        at = attrc[pl.ds(e, _L)][0]
                        for j in range(nsub):
                            sl = pl.ds(j * _L, _L)
                            m = rows[e, sl] + eemb_v[at, sl]
                            acc_s[ldst, sl] = acc_s[ldst, sl] + m
                            acc_q[ldst, sl] = acc_q[ldst, sl] + m * m
                            acc_mx[ldst, sl] = jnp.maximum(acc_mx[ldst, sl], m)
                            acc_mn[ldst, sl] = jnp.minimum(acc_mn[ldst, sl], m)
                        return ecarry

                    lax.fori_loop(lo, hi, edge_body, 0)
                    return carry

                lax.fori_loop(0, nrounds, round_body, 0)
                pltpu.sync_copy(acc_s, s_hbm.at[pl.ds(nb, _G)])
                pltpu.sync_copy(acc_q, q_hbm.at[pl.ds(nb, _G)])
                pltpu.sync_copy(acc_mx, mx_hbm.at[pl.ds(nb, _G)])
                pltpu.sync_copy(acc_mn, mn_hbm.at[pl.ds(nb, _G)])

    out = jax.ShapeDtypeStruct((n_pad, D), jnp.float32)
    fn = pl.kernel(
        body,
        out_type=(out, out, out, out),
        mesh=mesh,
        scratch_types=[
            pltpu.VMEM((eemb.shape[0], D), jnp.float32),
            pltpu.VMEM((nb_len,), jnp.int32),
            pltpu.VMEM((_CE,), jnp.int32),
            pltpu.VMEM((_CE + _L,), jnp.int32),
            pltpu.VMEM((_CE + _L,), jnp.int32),
            pltpu.VMEM((_CE, D), jnp.float32),
            pltpu.VMEM((_G, D), jnp.float32),
            pltpu.VMEM((_G, D), jnp.float32),
            pltpu.VMEM((_G, D), jnp.float32),
            pltpu.VMEM((_G, D), jnp.float32),
            pltpu.SemaphoreType.DMA,
        ],
    )
    return fn(h, eemb, ssrc_p, sattr_p, sdst_p, bounds)


def _tc_layer(h, s, q, mx, mn, rs0, rs1, sn2, W, b2):
    N, D = h.shape
    BN = 400
    grid = (N // BN,)

    def body(h_r, s_r, q_r, mx_r, mn_r, r0_r, r1_r, sn_r, w_r, b_r, o_r):
        deg = (r1_r[...] - r0_r[...]).astype(jnp.float32)
        degc = jnp.maximum(deg, 1.0)
        logd = jnp.log(deg + 1.0)
        amp = logd / _DELTA
        att = _DELTA / jnp.maximum(logd, 1e-6)
        hv = h_r[...]
        mean = s_r[...] / degc
        std = jnp.sqrt(jnp.maximum(q_r[...] / degc - mean * mean, 0.0) + 1e-5)
        nonempty = deg > 0.0
        mxv = jnp.where(nonempty, mx_r[...], 0.0)
        mnv = jnp.where(nonempty, mn_r[...], 0.0)
        agg = jnp.concatenate([mean, mnv, mxv, std], axis=1)
        z = jnp.concatenate([hv, agg, agg * amp, agg * att], axis=1)
        o = jnp.dot(z, w_r[...], precision=_PREC,
                    preferred_element_type=jnp.float32) + b_r[...]
        o = jnp.maximum(o, 0.0) + hv
        o_r[...] = o * sn_r[...]

    row_spec = pl.BlockSpec((BN, D), lambda i: (i, 0))
    col_spec = pl.BlockSpec((BN, 1), lambda i: (i, 0))
    return pl.pallas_call(
        body,
        grid=grid,
        in_specs=[
            row_spec, row_spec, row_spec, row_spec, row_spec,
            col_spec, col_spec, col_spec,
            pl.BlockSpec(W.shape, lambda i: (0, 0)),
            pl.BlockSpec((1, D), lambda i: (0, 0)),
        ],
        out_specs=row_spec,
        out_shape=jax.ShapeDtypeStruct((N, D), jnp.float32),
    )(h, s, q, mx, mn, rs0, rs1, sn2, W, b2)


def _tc_pool(h, batch2, Wm1, bm1_2, Wm2, bm2_2):
    N, D = h.shape
    BN = 400
    g = N // BN

    def body(h_r, b_r, w1_r, b1_r, w2_r, b2_r, o_r, acc):
        i = pl.program_id(0)

        @pl.when(i == 0)
        def _():
            acc[...] = jnp.zeros_like(acc)

        oh = (b_r[...] == lax.broadcasted_iota(jnp.int32, (BN, _B), 1))
        acc[...] += lax.dot_general(
            oh.astype(jnp.float32), h_r[...],
            (((0,), (0,)), ((), ())), precision=_PREC,
            preferred_element_type=jnp.float32)

        @pl.when(i == g - 1)
        def _():
            p = acc[...]
            hid = jnp.maximum(
                jnp.dot(p, w1_r[...], precision=_PREC,
                        preferred_element_type=jnp.float32)
                + b1_r[...], 0.0)
            o_r[...] = (jnp.dot(hid, w2_r[...], precision=_PREC,
                                preferred_element_type=jnp.float32) + b2_r[...])

    return pl.pallas_call(
        body,
        grid=(g,),
        in_specs=[
            pl.BlockSpec((BN, D), lambda i: (i, 0)),
            pl.BlockSpec((BN, 1), lambda i: (i, 0)),
            pl.BlockSpec(Wm1.shape, lambda i: (0, 0)),
            pl.BlockSpec((1, D), lambda i: (0, 0)),
            pl.BlockSpec(Wm2.shape, lambda i: (0, 0)),
            pl.BlockSpec((1, 1), lambda i: (0, 0)),
        ],
        out_specs=pl.BlockSpec((_B, 1), lambda i: (0, 0)),
        out_shape=jax.ShapeDtypeStruct((_B, 1), jnp.float32),
        scratch_shapes=[pltpu.VMEM((_B, D), jnp.float32)],
    )(h, batch2, Wm1, bm1_2, Wm2, bm2_2)


def kernel(X_n, snorm, Eemb0, Eemb1, Eemb2, W0, b0, W1, b1, W2, b2,
           Wm1, bm1, Wm2, bm2, edge_index, edge_attr, batch):
    N, D = X_n.shape
    E = edge_index.shape[1]
    src = edge_index[0]
    dst = edge_index[1]

    sdst, perm = lax.sort_key_val(dst, jnp.arange(E, dtype=jnp.int32))
    ssrc = src[perm]
    sattr = edge_attr[perm]
    row_start = jnp.searchsorted(sdst, jnp.arange(N + 1)).astype(jnp.int32)

    n_chunks = -(-N // _G)
    n_pad = n_chunks * _G
    nb_len = -(-(n_chunks + _L) // _L) * _L
    cidx = jnp.minimum(jnp.arange(n_chunks + 1) * _G, N)
    bounds = jnp.concatenate(
        [row_start[cidx],
         jnp.full((nb_len - (n_chunks + 1),), E, jnp.int32)])
    pad_i = jnp.zeros((_CE + 8,), jnp.int32)
    ssrc_p = jnp.concatenate([ssrc, pad_i])
    sattr_p = jnp.concatenate([sattr, pad_i])
    sdst_p = jnp.concatenate([sdst, pad_i])

    rs0 = row_start[:N].reshape(N, 1)
    rs1 = row_start[1:N + 1].reshape(N, 1)
    sn2 = snorm.reshape(N, 1)

    h = X_n
    for Eemb, W, b in ((Eemb0, W0, b0), (Eemb1, W1, b1), (Eemb2, W2, b2)):
        s, q, mx, mn = _sc_edge_stage(h, Eemb, ssrc_p, sattr_p, sdst_p,
                                      bounds, n_chunks, n_pad)
        h = _tc_layer(h, s[:N], q[:N], mx[:N], mn[:N], rs0, rs1, sn2,
                      W, b.reshape(1, D))

    y = _tc_pool(h, batch.reshape(N, 1), Wm1, bm1.reshape(1, D),
                 Wm2, bm2.reshape(1, 1))
    return y.reshape(_B)


# speedup vs baseline: 2.6513x; 1.0766x over previous
import math

import jax
import jax.numpy as jnp
from jax import lax
from jax.experimental import pallas as pl
from jax.experimental.pallas import tpu as pltpu
from jax.experimental.pallas import tpu_sc as plsc

_DEG_HIST = (0.0, 500.0, 1500.0, 2500.0, 2500.0, 1500.0, 1000.0, 500.0)
_DELTA = sum(h * math.log(i + 1.0) for i, h in enumerate(_DEG_HIST)) / sum(_DEG_HIST)

_B = 64
_PREC = lax.Precision.HIGHEST
_G = 128
_CE = 128
_L = 16


def _sc_edge_stage(h, eemb, ssrc_p, sattr_p, sdst_p, bounds, n_chunks, n_pad):
    D = h.shape[1]
    nsub = D // _L
    info = plsc.get_sparse_core_info()
    NC, NS = info.num_cores, info.num_subcores
    NW = NC * NS
    kmax = -(-n_chunks // NW)
    nb_len = bounds.shape[0]
    mesh = plsc.VectorSubcoreMesh(core_axis_name="c", subcore_axis_name="s")

    def body(h_hbm, eemb_hbm, ssrc_hbm, sattr_hbm, sdst_hbm, bnd_hbm,
             s_hbm, q_hbm, mx_hbm, mn_hbm,
             eemb_v, bnd_v, srcc0, attrc0, dstc0, rows0,
             srcc1, attrc1, dstc1, rows1,
             acc_s, acc_q, acc_mx, acc_mn, sem0, sem1):
        wid = lax.axis_index("s") * NC + lax.axis_index("c")
        pltpu.sync_copy(eemb_hbm, eemb_v)
        pltpu.sync_copy(bnd_hbm, bnd_v)
        zero = jnp.zeros((_L,), jnp.float32)
        neg = jnp.full((_L,), -jnp.inf, jnp.float32)
        pos = jnp.full((_L,), jnp.inf, jnp.float32)

        for k in range(kmax):
            c = wid + NW * k

            @pl.when(c < n_chunks)
            def _():
                nb = c * _G
                bv = bnd_v[pl.ds(c, _L)]
                e0 = bv[0]
                e1 = bv[1]

                def init_row(r, carry):
                    for j in range(nsub):
                        sl = pl.ds(j * _L, _L)
                        acc_s[r, sl] = zero
                        acc_q[r, sl] = zero
                        acc_mx[r, sl] = neg
                        acc_mn[r, sl] = pos
                    return carry

                lax.fori_loop(0, _G, init_row, 0)

                eb0 = (e0 // 8) * 8
                nrounds = (e1 - eb0 + _CE - 1) // _CE
                npairs = (nrounds + 1) // 2

                def issue(i, srcc, attrc, dstc, rows, sem):
                    eb = eb0 + i * _CE
                    pltpu.sync_copy(ssrc_hbm.at[pl.ds(eb, _CE)],
                                    srcc.at[pl.ds(0, _CE)])
                    pltpu.sync_copy(sattr_hbm.at[pl.ds(eb, _CE)],
                                    attrc.at[pl.ds(0, _CE)])
                    pltpu.sync_copy(sdst_hbm.at[pl.ds(eb, _CE)],
                                    dstc.at[pl.ds(0, _CE)])
                    pltpu.async_copy(h_hbm.at[srcc], rows, sem)

                def compute(i, attrc, dstc, rows):
                    eb = eb0 + i * _CE
                    lo = jnp.maximum(e0 - eb, 0)
                    hi = jnp.minimum(e1 - eb, _CE)

                    def edge_body(e, ecarry):
                        ldst = dstc[pl.ds(e, _L)][0] - nb
                        at = attrc[pl.ds(e, _L)][0]
                        for j in range(nsub):
                            sl = pl.ds(j * _L, _L)
                            m = rows[e, sl] + eemb_v[at, sl]
                            acc_s[ldst, sl] = acc_s[ldst, sl] + m
                            acc_q[ldst, sl] = acc_q[ldst, sl] + m * m
                            acc_mx[ldst, sl] = jnp.maximum(acc_mx[ldst, sl], m)
                            acc_mn[ldst, sl] = jnp.minimum(acc_mn[ldst, sl], m)
                        return ecarry

                    lax.fori_loop(lo, hi, edge_body, 0)

                @pl.when(nrounds > 0)
                def _():
                    issue(0, srcc0, attrc0, dstc0, rows0, sem0)

                def pair_body(p, carry):
                    i0 = 2 * p
                    i1 = 2 * p + 1

                    @pl.when(i1 < nrounds)
                    def _():
                        issue(i1, srcc1, attrc1, dstc1, rows1, sem1)

                    pltpu.make_async_copy(h_hbm.at[srcc0], rows0, sem0).wait()
                    compute(i0, attrc0, dstc0, rows0)

                    @pl.when(i1 + 1 < nrounds)
                    def _():
                        issue(i1 + 1, srcc0, attrc0, dstc0, rows0, sem0)

                    @pl.when(i1 < nrounds)
                    def _():
                        pltpu.make_async_copy(
                            h_hbm.at[srcc1], rows1, sem1).wait()
                        compute(i1, attrc1, dstc1, rows1)

                    return carry

                lax.fori_loop(0, npairs, pair_body, 0)
                pltpu.sync_copy(acc_s, s_hbm.at[pl.ds(nb, _G)])
                pltpu.sync_copy(acc_q, q_hbm.at[pl.ds(nb, _G)])
                pltpu.sync_copy(acc_mx, mx_hbm.at[pl.ds(nb, _G)])
                pltpu.sync_copy(acc_mn, mn_hbm.at[pl.ds(nb, _G)])

    out = jax.ShapeDtypeStruct((n_pad, D), jnp.float32)
    fn = pl.kernel(
        body,
        out_type=(out, out, out, out),
        mesh=mesh,
        scratch_types=[
            pltpu.VMEM((eemb.shape[0], D), jnp.float32),
            pltpu.VMEM((nb_len,), jnp.int32),
            pltpu.VMEM((_CE,), jnp.int32),
            pltpu.VMEM((_CE + _L,), jnp.int32),
            pltpu.VMEM((_CE + _L,), jnp.int32),
            pltpu.VMEM((_CE, D), jnp.float32),
            pltpu.VMEM((_CE,), jnp.int32),
            pltpu.VMEM((_CE + _L,), jnp.int32),
            pltpu.VMEM((_CE + _L,), jnp.int32),
            pltpu.VMEM((_CE, D), jnp.float32),
            pltpu.VMEM((_G, D), jnp.float32),
            pltpu.VMEM((_G, D), jnp.float32),
            pltpu.VMEM((_G, D), jnp.float32),
            pltpu.VMEM((_G, D), jnp.float32),
            pltpu.SemaphoreType.DMA,
            pltpu.SemaphoreType.DMA,
        ],
    )
    return fn(h, eemb, ssrc_p, sattr_p, sdst_p, bounds)


def _tc_layer(h, s, q, mx, mn, rs0, rs1, sn2, W, b2):
    N, D = h.shape
    BN = 400
    grid = (N // BN,)

    def body(h_r, s_r, q_r, mx_r, mn_r, r0_r, r1_r, sn_r, w_r, b_r, o_r):
        deg = (r1_r[...] - r0_r[...]).astype(jnp.float32)
        degc = jnp.maximum(deg, 1.0)
        logd = jnp.log(deg + 1.0)
        amp = logd / _DELTA
        att = _DELTA / jnp.maximum(logd, 1e-6)
        hv = h_r[...]
        mean = s_r[...] / degc
        std = jnp.sqrt(jnp.maximum(q_r[...] / degc - mean * mean, 0.0) + 1e-5)
        nonempty = deg > 0.0
        mxv = jnp.where(nonempty, mx_r[...], 0.0)
        mnv = jnp.where(nonempty, mn_r[...], 0.0)
        agg = jnp.concatenate([mean, mnv, mxv, std], axis=1)
        z = jnp.concatenate([hv, agg, agg * amp, agg * att], axis=1)
        o = jnp.dot(z, w_r[...], precision=_PREC,
                    preferred_element_type=jnp.float32) + b_r[...]
        o = jnp.maximum(o, 0.0) + hv
        o_r[...] = o * sn_r[...]

    row_spec = pl.BlockSpec((BN, D), lambda i: (i, 0))
    col_spec = pl.BlockSpec((BN, 1), lambda i: (i, 0))
    return pl.pallas_call(
        body,
        grid=grid,
        in_specs=[
            row_spec, row_spec, row_spec, row_spec, row_spec,
            col_spec, col_spec, col_spec,
            pl.BlockSpec(W.shape, lambda i: (0, 0)),
            pl.BlockSpec((1, D), lambda i: (0, 0)),
        ],
        out_specs=row_spec,
        out_shape=jax.ShapeDtypeStruct((N, D), jnp.float32),
    )(h, s, q, mx, mn, rs0, rs1, sn2, W, b2)


def _tc_pool(h, batch2, Wm1, bm1_2, Wm2, bm2_2):
    N, D = h.shape
    BN = 400
    g = N // BN

    def body(h_r, b_r, w1_r, b1_r, w2_r, b2_r, o_r, acc):
        i = pl.program_id(0)

        @pl.when(i == 0)
        def _():
            acc[...] = jnp.zeros_like(acc)

        oh = (b_r[...] == lax.broadcasted_iota(jnp.int32, (BN, _B), 1))
        acc[...] += lax.dot_general(
            oh.astype(jnp.float32), h_r[...],
            (((0,), (0,)), ((), ())), precision=_PREC,
            preferred_element_type=jnp.float32)

        @pl.when(i == g - 1)
        def _():
            p = acc[...]
            hid = jnp.maximum(
                jnp.dot(p, w1_r[...], precision=_PREC,
                        preferred_element_type=jnp.float32)
                + b1_r[...], 0.0)
            o_r[...] = (jnp.dot(hid, w2_r[...], precision=_PREC,
                                preferred_element_type=jnp.float32) + b2_r[...])

    return pl.pallas_call(
        body,
        grid=(g,),
        in_specs=[
            pl.BlockSpec((BN, D), lambda i: (i, 0)),
            pl.BlockSpec((BN, 1), lambda i: (i, 0)),
            pl.BlockSpec(Wm1.shape, lambda i: (0, 0)),
            pl.BlockSpec((1, D), lambda i: (0, 0)),
            pl.BlockSpec(Wm2.shape, lambda i: (0, 0)),
            pl.BlockSpec((1, 1), lambda i: (0, 0)),
        ],
        out_specs=pl.BlockSpec((_B, 1), lambda i: (0, 0)),
        out_shape=jax.ShapeDtypeStruct((_B, 1), jnp.float32),
        scratch_shapes=[pltpu.VMEM((_B, D), jnp.float32)],
    )(h, batch2, Wm1, bm1_2, Wm2, bm2_2)


def kernel(X_n, snorm, Eemb0, Eemb1, Eemb2, W0, b0, W1, b1, W2, b2,
           Wm1, bm1, Wm2, bm2, edge_index, edge_attr, batch):
    N, D = X_n.shape
    E = edge_index.shape[1]
    src = edge_index[0]
    dst = edge_index[1]

    sdst, perm = lax.sort_key_val(dst, jnp.arange(E, dtype=jnp.int32))
    ssrc = src[perm]
    sattr = edge_attr[perm]
    row_start = jnp.searchsorted(sdst, jnp.arange(N + 1)).astype(jnp.int32)

    n_chunks = -(-N // _G)
    n_pad = n_chunks * _G
    nb_len = -(-(n_chunks + _L) // _L) * _L
    cidx = jnp.minimum(jnp.arange(n_chunks + 1) * _G, N)
    bounds = jnp.concatenate(
        [row_start[cidx],
         jnp.full((nb_len - (n_chunks + 1),), E, jnp.int32)])
    pad_i = jnp.zeros((_CE + 8,), jnp.int32)
    ssrc_p = jnp.concatenate([ssrc, pad_i])
    sattr_p = jnp.concatenate([sattr, pad_i])
    sdst_p = jnp.concatenate([sdst, pad_i])

    rs0 = row_start[:N].reshape(N, 1)
    rs1 = row_start[1:N + 1].reshape(N, 1)
    sn2 = snorm.reshape(N, 1)

    h = X_n
    for Eemb, W, b in ((Eemb0, W0, b0), (Eemb1, W1, b1), (Eemb2, W2, b2)):
        s, q, mx, mn = _sc_edge_stage(h, Eemb, ssrc_p, sattr_p, sdst_p,
                                      bounds, n_chunks, n_pad)
        h = _tc_layer(h, s[:N], q[:N], mx[:N], mn[:N], rs0, rs1, sn2,
                      W, b.reshape(1, D))

    y = _tc_pool(h, batch.reshape(N, 1), Wm1, bm1.reshape(1, D),
                 Wm2, bm2.reshape(1, 1))
    return y.reshape(_B)
